# baseline (speedup 1.0000x reference)
import functools

import jax
import jax.numpy as jnp
from jax import lax
from jax.experimental import pallas as pl
from jax.experimental.pallas import tpu as pltpu
from jax.experimental.pallas import tpu_sc as plsc

NUM_FIELDS = 26
FIELD_DIM = 40000
BATCH = 16384
NC = 2
NS = 16
B_PER_CORE = BATCH // NC
B_PER_TILE = B_PER_CORE // NS
L = 16
MAIN_NROW = 8120
TAB_ROWS = 320


def _sc_body(xT, table, tail, bias, out, tab_v, idx_v, part_v, tmp_v, out_v,
             bias_v, shared_p):
    c = lax.axis_index("c")
    s = lax.axis_index("s")
    base_b = c * B_PER_CORE

    pltpu.sync_copy(bias, bias_v)

    def _do_field(f, first):
        pltpu.sync_copy(xT.at[f, pl.ds(base_b, B_PER_CORE)], idx_v)
        flat0 = f * FIELD_DIM
        row_start = jnp.minimum((flat0 >> 10) << 3, MAIN_NROW - TAB_ROWS)
        row_start = pl.multiple_of(row_start, 8)
        rem = flat0 - (row_start << 7)
        pltpu.sync_copy(table.at[pl.ds(row_start, TAB_ROWS)],
                        tab_v.at[pl.ds(0, TAB_ROWS)])

        @pl.when(f == NUM_FIELDS - 1)
        def _load_tail():
            pltpu.sync_copy(tail, tab_v.at[pl.ds(TAB_ROWS - 3, 8)])

        if first:
            @pl.loop(0, B_PER_CORE // L, unroll=4)
            def _gather(j):
                sl = pl.ds(j * L, L)
                w = idx_v[sl] + rem
                part_v[sl] = plsc.load_gather(tab_v, [w >> 7, w & 127])
        else:
            @pl.loop(0, B_PER_CORE // L, unroll=4)
            def _gather(j):
                sl = pl.ds(j * L, L)
                w = idx_v[sl] + rem
                part_v[sl] = part_v[sl] + plsc.load_gather(tab_v,
                                                           [w >> 7, w & 127])

    _do_field(s, True)

    @pl.when(s < NUM_FIELDS - NS)
    def _second_field():
        _do_field(s + NS, False)

    pltpu.sync_copy(part_v, shared_p.at[pl.ds(s * B_PER_CORE, B_PER_CORE)])
    plsc.subcore_barrier()

    bias_vec = bias_v[...]

    @pl.loop(0, B_PER_TILE // L)
    def _init(j):
        out_v[pl.ds(j * L, L)] = bias_vec

    for t in range(NS):
        pltpu.sync_copy(
            shared_p.at[pl.ds(t * B_PER_CORE + s * B_PER_TILE, B_PER_TILE)],
            tmp_v)

        @pl.loop(0, B_PER_TILE // L)
        def _acc(j):
            sl = pl.ds(j * L, L)
            out_v[sl] = out_v[sl] + tmp_v[sl]

    pltpu.sync_copy(out_v, out.at[pl.ds(base_b + s * B_PER_TILE, B_PER_TILE)])


_sc_kernel = functools.partial(
    pl.kernel,
    out_type=jax.ShapeDtypeStruct((BATCH,), jnp.float32),
    mesh=plsc.VectorSubcoreMesh(core_axis_name="c", subcore_axis_name="s",
                                num_cores=NC, num_subcores=NS),
    scratch_types=[
        pltpu.VMEM((TAB_ROWS + 8, 128), jnp.float32),
        pltpu.VMEM((B_PER_CORE,), jnp.int32),
        pltpu.VMEM((B_PER_CORE,), jnp.float32),
        pltpu.VMEM((B_PER_TILE,), jnp.float32),
        pltpu.VMEM((B_PER_TILE,), jnp.float32),
        pltpu.VMEM((L,), jnp.float32),
        pltpu.VMEM_SHARED((NS * B_PER_CORE,), jnp.float32),
    ],
    compiler_params=pltpu.CompilerParams(needs_layout_passes=False),
)(_sc_body)


@jax.jit
def kernel(x, fc_weight, bias):
    xT = x.astype(jnp.int32).T
    fcw = fc_weight.astype(jnp.float32)
    table = fcw[:MAIN_NROW * 128].reshape(MAIN_NROW, 128)
    tail = fcw[1038976:1040000].reshape(8, 128)
    bias16 = jnp.broadcast_to(bias.astype(jnp.float32), (L,))
    out = _sc_kernel(xT, table, tail, bias16)
    return out.reshape(BATCH, 1)


# speedup vs baseline: 1.0036x; 1.0036x over previous
import functools

import jax
import jax.numpy as jnp
from jax import lax
from jax.experimental import pallas as pl
from jax.experimental.pallas import tpu as pltpu
from jax.experimental.pallas import tpu_sc as plsc

NUM_FIELDS = 26
FIELD_DIM = 40000
BATCH = 16384
NC = 2
NS = 16
B_PER_CORE = BATCH // NC
B_PER_TILE = B_PER_CORE // NS
L = 16
TAB_NROW = 8128
TAB_ROWS = 320


def _sc_body(xT, table, bias, out, tab_v, idx_v, part_v, tmp_v, out_v,
             bias_v, shared_p):
    c = lax.axis_index("c")
    s = lax.axis_index("s")
    base_b = c * B_PER_CORE

    pltpu.sync_copy(bias, bias_v)

    def _do_field(f, first):
        pltpu.sync_copy(xT.at[f, pl.ds(base_b, B_PER_CORE)], idx_v)
        flat0 = f * FIELD_DIM
        row_start = jnp.minimum((flat0 >> 10) << 3, TAB_NROW - TAB_ROWS)
        row_start = pl.multiple_of(row_start, 8)
        rem = flat0 - (row_start << 7)
        pltpu.sync_copy(table.at[pl.ds(row_start, TAB_ROWS)], tab_v)

        if first:
            @pl.loop(0, B_PER_CORE // L, unroll=4)
            def _gather(j):
                sl = pl.ds(j * L, L)
                w = idx_v[sl] + rem
                part_v[sl] = plsc.load_gather(tab_v, [w >> 7, w & 127])
        else:
            @pl.loop(0, B_PER_CORE // L, unroll=4)
            def _gather(j):
                sl = pl.ds(j * L, L)
                w = idx_v[sl] + rem
                part_v[sl] = part_v[sl] + plsc.load_gather(tab_v,
                                                           [w >> 7, w & 127])

    _do_field(s, True)

    @pl.when(s < NUM_FIELDS - NS)
    def _second_field():
        _do_field(s + NS, False)

    pltpu.sync_copy(part_v, shared_p.at[pl.ds(s * B_PER_CORE, B_PER_CORE)])
    plsc.subcore_barrier()

    bias_vec = bias_v[...]

    @pl.loop(0, B_PER_TILE // L)
    def _init(j):
        out_v[pl.ds(j * L, L)] = bias_vec

    for t in range(NS):
        pltpu.sync_copy(
            shared_p.at[pl.ds(t * B_PER_CORE + s * B_PER_TILE, B_PER_TILE)],
            tmp_v)

        @pl.loop(0, B_PER_TILE // L)
        def _acc(j):
            sl = pl.ds(j * L, L)
            out_v[sl] = out_v[sl] + tmp_v[sl]

    pltpu.sync_copy(out_v, out.at[pl.ds(base_b + s * B_PER_TILE, B_PER_TILE)])


_sc_kernel = functools.partial(
    pl.kernel,
    out_type=jax.ShapeDtypeStruct((BATCH,), jnp.float32),
    mesh=plsc.VectorSubcoreMesh(core_axis_name="c", subcore_axis_name="s",
                                num_cores=NC, num_subcores=NS),
    scratch_types=[
        pltpu.VMEM((TAB_ROWS, 128), jnp.float32),
        pltpu.VMEM((B_PER_CORE,), jnp.int32),
        pltpu.VMEM((B_PER_CORE,), jnp.float32),
        pltpu.VMEM((B_PER_TILE,), jnp.float32),
        pltpu.VMEM((B_PER_TILE,), jnp.float32),
        pltpu.VMEM((L,), jnp.float32),
        pltpu.VMEM_SHARED((NS * B_PER_CORE,), jnp.float32),
    ],
    compiler_params=pltpu.CompilerParams(needs_layout_passes=False),
)(_sc_body)


@jax.jit
def kernel(x, fc_weight, bias):
    xT = x.astype(jnp.int32).T
    fcp = jnp.pad(fc_weight.astype(jnp.float32), ((0, 384), (0, 0)))
    table = fcp.reshape(TAB_NROW, 128)
    bias16 = jnp.broadcast_to(bias.astype(jnp.float32), (L,))
    out = _sc_kernel(xT, table, bias16)
    return out.reshape(BATCH, 1)


# speedup vs baseline: 1.1245x; 1.1204x over previous
import functools

import jax
import jax.numpy as jnp
from jax import lax
from jax.experimental import pallas as pl
from jax.experimental.pallas import tpu as pltpu
from jax.experimental.pallas import tpu_sc as plsc

NUM_FIELDS = 26
FIELD_DIM = 40000
BATCH = 16384
NC = 2
NS = 16
B_PER_CORE = BATCH // NC
B_PER_TILE = B_PER_CORE // NS
L = 16
TAB_NROW = 8128
TAB_ROWS = 320


def _sc_body(xT, table, bias, out, tab_v, idx_v, part_v, tmp_v, out_v,
             bias_v, shared_p):
    c = lax.axis_index("c")
    s = lax.axis_index("s")
    base_b = c * B_PER_CORE

    pltpu.sync_copy(bias, bias_v)

    def _do_field(f, first):
        with jax.named_scope("idx_dma"):
            pltpu.sync_copy(xT.at[f, pl.ds(base_b, B_PER_CORE)], idx_v)
        flat0 = f * FIELD_DIM
        row_start = jnp.minimum((flat0 >> 10) << 3, TAB_NROW - TAB_ROWS)
        row_start = pl.multiple_of(row_start, 8)
        rem = flat0 - (row_start << 7)
        with jax.named_scope("tab_dma"):
            pltpu.sync_copy(table.at[pl.ds(row_start, TAB_ROWS)], tab_v)

        with jax.named_scope("gather"):
            if first:
                @pl.loop(0, B_PER_CORE // L)
                def _gather(j):
                    sl = pl.ds(j * L, L)
                    w = idx_v[sl] + rem
                    part_v[sl] = plsc.load_gather(tab_v, [w >> 7, w & 127])
            else:
                @pl.loop(0, B_PER_CORE // L)
                def _gather(j):
                    sl = pl.ds(j * L, L)
                    w = idx_v[sl] + rem
                    part_v[sl] = part_v[sl] + plsc.load_gather(
                        tab_v, [w >> 7, w & 127])

    _do_field(s, True)

    @pl.when(s < NUM_FIELDS - NS)
    def _second_field():
        _do_field(s + NS, False)

    with jax.named_scope("publish"):
        pltpu.sync_copy(part_v,
                        shared_p.at[pl.ds(s * B_PER_CORE, B_PER_CORE)])
    with jax.named_scope("barrier"):
        plsc.subcore_barrier()

    bias_vec = bias_v[...]

    @pl.loop(0, B_PER_TILE // L)
    def _init(j):
        out_v[pl.ds(j * L, L)] = bias_vec

    with jax.named_scope("reduce"):
        for t in range(NS):
            pltpu.sync_copy(
                shared_p.at[pl.ds(t * B_PER_CORE + s * B_PER_TILE,
                                  B_PER_TILE)],
                tmp_v)

            @pl.loop(0, B_PER_TILE // L)
            def _acc(j):
                sl = pl.ds(j * L, L)
                out_v[sl] = out_v[sl] + tmp_v[sl]

    with jax.named_scope("out_dma"):
        pltpu.sync_copy(out_v,
                        out.at[pl.ds(base_b + s * B_PER_TILE, B_PER_TILE)])


_sc_kernel = functools.partial(
    pl.kernel,
    out_type=jax.ShapeDtypeStruct((BATCH,), jnp.float32),
    mesh=plsc.VectorSubcoreMesh(core_axis_name="c", subcore_axis_name="s",
                                num_cores=NC, num_subcores=NS),
    scratch_types=[
        pltpu.VMEM((TAB_ROWS, 128), jnp.float32),
        pltpu.VMEM((B_PER_CORE,), jnp.int32),
        pltpu.VMEM((B_PER_CORE,), jnp.float32),
        pltpu.VMEM((B_PER_TILE,), jnp.float32),
        pltpu.VMEM((B_PER_TILE,), jnp.float32),
        pltpu.VMEM((L,), jnp.float32),
        pltpu.VMEM_SHARED((NS * B_PER_CORE,), jnp.float32),
    ],
    compiler_params=pltpu.CompilerParams(needs_layout_passes=False),
)(_sc_body)


@jax.jit
def kernel(x, fc_weight, bias):
    xT = x.astype(jnp.int32).T
    fcp = jnp.pad(fc_weight.astype(jnp.float32), ((0, 384), (0, 0)))
    table = fcp.reshape(TAB_NROW, 128)
    bias16 = jnp.broadcast_to(bias.astype(jnp.float32), (L,))
    out = _sc_kernel(xT, table, bias16)
    return out.reshape(BATCH, 1)


# speedup vs baseline: 1.2953x; 1.1519x over previous
import functools

import jax
import jax.numpy as jnp
from jax import lax
from jax.experimental import pallas as pl
from jax.experimental.pallas import tpu as pltpu
from jax.experimental.pallas import tpu_sc as plsc

NUM_FIELDS = 26
FIELD_DIM = 40000
BATCH = 16384
NC = 2
NS = 16
B_PER_CORE = BATCH // NC
B_PER_TILE = B_PER_CORE // NS
L = 16
TAB_NROW = 8128
TAB_ROWS = 320


def _tab_window(f):
    flat0 = f * FIELD_DIM
    row_start = jnp.minimum((flat0 >> 10) << 3, TAB_NROW - TAB_ROWS)
    row_start = pl.multiple_of(row_start, 8)
    rem = flat0 - (row_start << 7)
    return row_start, rem


def _sc_body(xT, table, bias, out, tab_v, idx1_v, idx2_v, part_v, tmp16_v,
             out_v, bias_v, shared_p, sem_i1, sem_i2, sem_t, sem_p):
    c = lax.axis_index("c")
    s = lax.axis_index("s")
    base_b = c * B_PER_CORE
    two = s < NUM_FIELDS - NS
    f1 = s
    f2 = jnp.minimum(s + NS, NUM_FIELDS - 1)

    with jax.named_scope("fire_dmas"):
        d_i1 = pltpu.async_copy(xT.at[f1, pl.ds(base_b, B_PER_CORE)], idx1_v,
                                sem_i1)
        row1, rem1 = _tab_window(f1)
        d_t1 = pltpu.async_copy(table.at[pl.ds(row1, TAB_ROWS)], tab_v, sem_t)
        d_i2 = pltpu.async_copy(xT.at[f2, pl.ds(base_b, B_PER_CORE)], idx2_v,
                                sem_i2)
        pltpu.sync_copy(bias, bias_v)
        d_i1.wait()
        d_t1.wait()

    with jax.named_scope("gather1"):
        @pl.loop(0, B_PER_CORE // L)
        def _gather1(j):
            sl = pl.ds(j * L, L)
            w = idx1_v[sl] + rem1
            part_v[sl] = plsc.load_gather(tab_v, [w >> 7, w & 127])

    with jax.named_scope("drain_i2"):
        d_i2.wait()

    @pl.when(two)
    def _second_field():
        row2, rem2 = _tab_window(f2)
        with jax.named_scope("tab2_dma"):
            pltpu.sync_copy(table.at[pl.ds(row2, TAB_ROWS)], tab_v)

        with jax.named_scope("gather2"):
            @pl.loop(0, B_PER_CORE // L)
            def _gather2(j):
                sl = pl.ds(j * L, L)
                w = idx2_v[sl] + rem2
                plsc.addupdate(part_v.at[sl],
                               plsc.load_gather(tab_v, [w >> 7, w & 127]))

    with jax.named_scope("publish"):
        descs = []
        for r in range(NS):
            descs.append(pltpu.async_copy(
                part_v.at[pl.ds(r * B_PER_TILE, B_PER_TILE)],
                shared_p.at[pl.ds(r * B_PER_CORE + s * B_PER_TILE,
                                  B_PER_TILE)],
                sem_p))
        for d in descs:
            d.wait()
    with jax.named_scope("barrier"):
        plsc.subcore_barrier()

    with jax.named_scope("reduce"):
        pltpu.sync_copy(shared_p.at[pl.ds(s * B_PER_CORE, B_PER_CORE)],
                        tmp16_v)
        bias_vec = bias_v[...]

        @pl.loop(0, B_PER_TILE // L)
        def _acc(j):
            acc = bias_vec
            for t in range(NS):
                acc = acc + tmp16_v[pl.ds(t * B_PER_TILE + j * L, L)]
            out_v[pl.ds(j * L, L)] = acc

    with jax.named_scope("out_dma"):
        pltpu.sync_copy(out_v,
                        out.at[pl.ds(base_b + s * B_PER_TILE, B_PER_TILE)])


_sc_kernel = functools.partial(
    pl.kernel,
    out_type=jax.ShapeDtypeStruct((BATCH,), jnp.float32),
    mesh=plsc.VectorSubcoreMesh(core_axis_name="c", subcore_axis_name="s",
                                num_cores=NC, num_subcores=NS),
    scratch_types=[
        pltpu.VMEM((TAB_ROWS, 128), jnp.float32),
        pltpu.VMEM((B_PER_CORE,), jnp.int32),
        pltpu.VMEM((B_PER_CORE,), jnp.int32),
        pltpu.VMEM((B_PER_CORE,), jnp.float32),
        pltpu.VMEM((B_PER_CORE,), jnp.float32),
        pltpu.VMEM((B_PER_TILE,), jnp.float32),
        pltpu.VMEM((L,), jnp.float32),
        pltpu.VMEM_SHARED((NS * B_PER_CORE,), jnp.float32),
        pltpu.SemaphoreType.DMA,
        pltpu.SemaphoreType.DMA,
        pltpu.SemaphoreType.DMA,
        pltpu.SemaphoreType.DMA,
    ],
    compiler_params=pltpu.CompilerParams(needs_layout_passes=False),
)(_sc_body)


@jax.jit
def kernel(x, fc_weight, bias):
    xT = x.astype(jnp.int32).T
    fcp = jnp.pad(fc_weight.astype(jnp.float32), ((0, 384), (0, 0)))
    table = fcp.reshape(TAB_NROW, 128)
    bias16 = jnp.broadcast_to(bias.astype(jnp.float32), (L,))
    out = _sc_kernel(xT, table, bias16)
    return out.reshape(BATCH, 1)


# speedup vs baseline: 1.2976x; 1.0018x over previous
import functools

import jax
import jax.numpy as jnp
from jax import lax
from jax.experimental import pallas as pl
from jax.experimental.pallas import tpu as pltpu
from jax.experimental.pallas import tpu_sc as plsc

NUM_FIELDS = 26
FIELD_DIM = 40000
BATCH = 16384
NC = 2
NS = 16
B_PER_CORE = BATCH // NC
B_PER_TILE = B_PER_CORE // NS
L = 16
TAB_NROW = 8128
TAB_ROWS = 320


def _tab_window(f):
    flat0 = f * FIELD_DIM
    row_start = jnp.minimum((flat0 >> 10) << 3, TAB_NROW - TAB_ROWS)
    row_start = pl.multiple_of(row_start, 8)
    rem = flat0 - (row_start << 7)
    return row_start, rem


def _sc_body(xT, table, bias, out, tab_v, tab2_v, idx1_v, idx2_v, part_v,
             tmp16_v, out_v, bias_v, shared_p, sem_i1, sem_i2, sem_t, sem_t2,
             sem_p):
    c = lax.axis_index("c")
    s = lax.axis_index("s")
    base_b = c * B_PER_CORE
    two = s < NUM_FIELDS - NS
    f1 = s
    f2 = jnp.minimum(s + NS, NUM_FIELDS - 1)

    with jax.named_scope("fire_dmas"):
        d_i1 = pltpu.async_copy(xT.at[f1, pl.ds(base_b, B_PER_CORE)], idx1_v,
                                sem_i1)
        row1, rem1 = _tab_window(f1)
        d_t1 = pltpu.async_copy(table.at[pl.ds(row1, TAB_ROWS)], tab_v, sem_t)
        d_i2 = pltpu.async_copy(xT.at[f2, pl.ds(base_b, B_PER_CORE)], idx2_v,
                                sem_i2)
        row2, rem2 = _tab_window(f2)
        d_t2 = pltpu.async_copy(table.at[pl.ds(row2, TAB_ROWS)], tab2_v,
                                sem_t2)
        pltpu.sync_copy(bias, bias_v)
        d_i1.wait()
        d_t1.wait()

    with jax.named_scope("gather1"):
        @pl.loop(0, B_PER_CORE // L)
        def _gather1(j):
            sl = pl.ds(j * L, L)
            w = idx1_v[sl] + rem1
            part_v[sl] = plsc.load_gather(tab_v, [w >> 7, w & 127])

    with jax.named_scope("drain2"):
        d_i2.wait()
        d_t2.wait()

    @pl.when(two)
    def _second_field():
        with jax.named_scope("gather2"):
            @pl.loop(0, B_PER_CORE // L)
            def _gather2(j):
                sl = pl.ds(j * L, L)
                w = idx2_v[sl] + rem2
                plsc.addupdate(part_v.at[sl],
                               plsc.load_gather(tab2_v, [w >> 7, w & 127]))

    with jax.named_scope("publish"):
        descs = []
        for r in range(NS):
            descs.append(pltpu.async_copy(
                part_v.at[pl.ds(r * B_PER_TILE, B_PER_TILE)],
                shared_p.at[pl.ds(r * B_PER_CORE + s * B_PER_TILE,
                                  B_PER_TILE)],
                sem_p))
        for d in descs:
            d.wait()
    with jax.named_scope("barrier"):
        plsc.subcore_barrier()

    with jax.named_scope("reduce"):
        pltpu.sync_copy(shared_p.at[pl.ds(s * B_PER_CORE, B_PER_CORE)],
                        tmp16_v)
        bias_vec = bias_v[...]

        @pl.loop(0, B_PER_TILE // L)
        def _acc(j):
            acc = bias_vec
            for t in range(NS):
                acc = acc + tmp16_v[pl.ds(t * B_PER_TILE + j * L, L)]
            out_v[pl.ds(j * L, L)] = acc

    with jax.named_scope("out_dma"):
        pltpu.sync_copy(out_v,
                        out.at[pl.ds(base_b + s * B_PER_TILE, B_PER_TILE)])


_sc_kernel = functools.partial(
    pl.kernel,
    out_type=jax.ShapeDtypeStruct((BATCH,), jnp.float32),
    mesh=plsc.VectorSubcoreMesh(core_axis_name="c", subcore_axis_name="s",
                                num_cores=NC, num_subcores=NS),
    scratch_types=[
        pltpu.VMEM((TAB_ROWS, 128), jnp.float32),
        pltpu.VMEM((TAB_ROWS, 128), jnp.float32),
        pltpu.VMEM((B_PER_CORE,), jnp.int32),
        pltpu.VMEM((B_PER_CORE,), jnp.int32),
        pltpu.VMEM((B_PER_CORE,), jnp.float32),
        pltpu.VMEM((B_PER_CORE,), jnp.float32),
        pltpu.VMEM((B_PER_TILE,), jnp.float32),
        pltpu.VMEM((L,), jnp.float32),
        pltpu.VMEM_SHARED((NS * B_PER_CORE,), jnp.float32),
        pltpu.SemaphoreType.DMA,
        pltpu.SemaphoreType.DMA,
        pltpu.SemaphoreType.DMA,
        pltpu.SemaphoreType.DMA,
        pltpu.SemaphoreType.DMA,
    ],
    compiler_params=pltpu.CompilerParams(needs_layout_passes=False),
)(_sc_body)


@jax.jit
def kernel(x, fc_weight, bias):
    xT = x.astype(jnp.int32).T
    fcp = jnp.pad(fc_weight.astype(jnp.float32), ((0, 384), (0, 0)))
    table = fcp.reshape(TAB_NROW, 128)
    bias16 = jnp.broadcast_to(bias.astype(jnp.float32), (L,))
    out = _sc_kernel(xT, table, bias16)
    return out.reshape(BATCH, 1)


# speedup vs baseline: 1.3472x; 1.0382x over previous
import functools

import jax
import jax.numpy as jnp
from jax import lax
from jax.experimental import pallas as pl
from jax.experimental.pallas import tpu as pltpu
from jax.experimental.pallas import tpu_sc as plsc

NUM_FIELDS = 26
FIELD_DIM = 40000
BATCH = 16384
NC = 2
NS = 16
B_PER_CORE = BATCH // NC
B_PER_TILE = B_PER_CORE // NS
L = 16
TAB_NROW = 8128
TAB_ROWS = 320


def _tab_window(f):
    flat0 = f * FIELD_DIM
    row_start = jnp.minimum((flat0 >> 10) << 3, TAB_NROW - TAB_ROWS)
    row_start = pl.multiple_of(row_start, 8)
    rem = flat0 - (row_start << 7)
    return row_start, rem


def _sc_body(xT, table, bias, out, tab_v, tab2_v, idx1_v, idx2_v, part_v,
             tmp16_v, out_v, bias_v, shared_p, sem_i1, sem_i2, sem_t, sem_t2,
             sem_p):
    c = lax.axis_index("c")
    s = lax.axis_index("s")
    base_b = c * B_PER_CORE
    two = s < NUM_FIELDS - NS
    f1 = s
    f2 = jnp.minimum(s + NS, NUM_FIELDS - 1)

    with jax.named_scope("fire_dmas"):
        d_i1 = pltpu.async_copy(xT.at[f1, pl.ds(base_b, B_PER_CORE)], idx1_v,
                                sem_i1)
        row1, rem1 = _tab_window(f1)
        d_t1 = pltpu.async_copy(table.at[pl.ds(row1, TAB_ROWS)], tab_v, sem_t)
        pltpu.sync_copy(bias, bias_v)
        d_i1.wait()
        d_t1.wait()
        row2, rem2 = _tab_window(f2)
        d_i2 = pltpu.async_copy(xT.at[f2, pl.ds(base_b, B_PER_CORE)], idx2_v,
                                sem_i2)
        d_t2 = pltpu.async_copy(table.at[pl.ds(row2, TAB_ROWS)], tab2_v,
                                sem_t2)

    with jax.named_scope("gather1"):
        @pl.loop(0, B_PER_CORE // L)
        def _gather1(j):
            sl = pl.ds(j * L, L)
            w = idx1_v[sl] + rem1
            part_v[sl] = plsc.load_gather(tab_v, [w >> 7, w & 127])

    with jax.named_scope("drain2"):
        d_i2.wait()
        d_t2.wait()

    @pl.when(two)
    def _second_field():
        with jax.named_scope("gather2"):
            @pl.loop(0, B_PER_CORE // L)
            def _gather2(j):
                sl = pl.ds(j * L, L)
                w = idx2_v[sl] + rem2
                plsc.addupdate(part_v.at[sl],
                               plsc.load_gather(tab2_v, [w >> 7, w & 127]))

    with jax.named_scope("publish"):
        descs = []
        for r in range(NS):
            descs.append(pltpu.async_copy(
                part_v.at[pl.ds(r * B_PER_TILE, B_PER_TILE)],
                shared_p.at[pl.ds(r * B_PER_CORE + s * B_PER_TILE,
                                  B_PER_TILE)],
                sem_p))
        for d in descs:
            d.wait()
    with jax.named_scope("barrier"):
        plsc.subcore_barrier()

    with jax.named_scope("reduce"):
        pltpu.sync_copy(shared_p.at[pl.ds(s * B_PER_CORE, B_PER_CORE)],
                        tmp16_v)
        bias_vec = bias_v[...]

        @pl.loop(0, B_PER_TILE // L)
        def _acc(j):
            acc = bias_vec
            for t in range(NS):
                acc = acc + tmp16_v[pl.ds(t * B_PER_TILE + j * L, L)]
            out_v[pl.ds(j * L, L)] = acc

    with jax.named_scope("out_dma"):
        pltpu.sync_copy(out_v,
                        out.at[pl.ds(base_b + s * B_PER_TILE, B_PER_TILE)])


_sc_kernel = functools.partial(
    pl.kernel,
    out_type=jax.ShapeDtypeStruct((BATCH,), jnp.float32),
    mesh=plsc.VectorSubcoreMesh(core_axis_name="c", subcore_axis_name="s",
                                num_cores=NC, num_subcores=NS),
    scratch_types=[
        pltpu.VMEM((TAB_ROWS, 128), jnp.float32),
        pltpu.VMEM((TAB_ROWS, 128), jnp.float32),
        pltpu.VMEM((B_PER_CORE,), jnp.int32),
        pltpu.VMEM((B_PER_CORE,), jnp.int32),
        pltpu.VMEM((B_PER_CORE,), jnp.float32),
        pltpu.VMEM((B_PER_CORE,), jnp.float32),
        pltpu.VMEM((B_PER_TILE,), jnp.float32),
        pltpu.VMEM((L,), jnp.float32),
        pltpu.VMEM_SHARED((NS * B_PER_CORE,), jnp.float32),
        pltpu.SemaphoreType.DMA,
        pltpu.SemaphoreType.DMA,
        pltpu.SemaphoreType.DMA,
        pltpu.SemaphoreType.DMA,
        pltpu.SemaphoreType.DMA,
    ],
    compiler_params=pltpu.CompilerParams(needs_layout_passes=False),
)(_sc_body)


@jax.jit
def kernel(x, fc_weight, bias):
    xT = x.astype(jnp.int32).T
    fcp = jnp.pad(fc_weight.astype(jnp.float32), ((0, 384), (0, 0)))
    table = fcp.reshape(TAB_NROW, 128)
    bias16 = jnp.broadcast_to(bias.astype(jnp.float32), (L,))
    out = _sc_kernel(xT, table, bias16)
    return out.reshape(BATCH, 1)


# speedup vs baseline: 1.3762x; 1.0215x over previous
import functools

import jax
import jax.numpy as jnp
from jax import lax
from jax.experimental import pallas as pl
from jax.experimental.pallas import tpu as pltpu
from jax.experimental.pallas import tpu_sc as plsc

NUM_FIELDS = 26
FIELD_DIM = 40000
BATCH = 16384
NC = 2
NS = 16
B_PER_CORE = BATCH // NC
B_PER_TILE = B_PER_CORE // NS
L = 16
TAB_NROW = 8128
TAB_ROWS = 320


def _tab_window(f):
    flat0 = f * FIELD_DIM
    row_start = jnp.minimum((flat0 >> 10) << 3, TAB_NROW - TAB_ROWS)
    row_start = pl.multiple_of(row_start, 8)
    rem = flat0 - (row_start << 7)
    return row_start, rem


def _sc_body(xT, table, bias, out, tab_v, tab2_v, idx1_v, idx2_v, part_v,
             tmp16_v, out_v, bias_v, shared_p, sem_i1, sem_i2, sem_t, sem_t2,
             sem_p):
    c = lax.axis_index("c")
    s = lax.axis_index("s")
    base_b = c * B_PER_CORE
    two = s < NUM_FIELDS - NS
    f1 = s
    f2 = jnp.minimum(s + NS, NUM_FIELDS - 1)

    with jax.named_scope("fire_dmas"):
        d_i1 = pltpu.async_copy(xT.at[f1, pl.ds(base_b, B_PER_CORE)], idx1_v,
                                sem_i1)
        row1, rem1 = _tab_window(f1)
        d_t1 = pltpu.async_copy(table.at[pl.ds(row1, TAB_ROWS)], tab_v, sem_t)
        pltpu.sync_copy(bias, bias_v)
        d_i1.wait()
        d_t1.wait()
        row2, rem2 = _tab_window(f2)

        @pl.when(two)
        def _fire_second():
            pltpu.async_copy(xT.at[f2, pl.ds(base_b, B_PER_CORE)], idx2_v,
                             sem_i2)
            pltpu.async_copy(table.at[pl.ds(row2, TAB_ROWS)], tab2_v, sem_t2)

    with jax.named_scope("gather1"):
        @pl.loop(0, B_PER_CORE // L)
        def _gather1(j):
            sl = pl.ds(j * L, L)
            w = idx1_v[sl] + rem1
            part_v[sl] = plsc.load_gather(tab_v, [w >> 7, w & 127])

    @pl.when(two)
    def _second_field():
        with jax.named_scope("drain2"):
            pltpu.make_async_copy(xT.at[f2, pl.ds(base_b, B_PER_CORE)],
                                  idx2_v, sem_i2).wait()
            pltpu.make_async_copy(table.at[pl.ds(row2, TAB_ROWS)], tab2_v,
                                  sem_t2).wait()
        with jax.named_scope("gather2"):
            @pl.loop(0, B_PER_CORE // L)
            def _gather2(j):
                sl = pl.ds(j * L, L)
                w = idx2_v[sl] + rem2
                plsc.addupdate(part_v.at[sl],
                               plsc.load_gather(tab2_v, [w >> 7, w & 127]))

    with jax.named_scope("publish"):
        descs = []
        for r in range(NS):
            descs.append(pltpu.async_copy(
                part_v.at[pl.ds(r * B_PER_TILE, B_PER_TILE)],
                shared_p.at[pl.ds(r * B_PER_CORE + s * B_PER_TILE,
                                  B_PER_TILE)],
                sem_p))
        for d in descs:
            d.wait()
    with jax.named_scope("barrier"):
        plsc.subcore_barrier()

    with jax.named_scope("reduce"):
        pltpu.sync_copy(shared_p.at[pl.ds(s * B_PER_CORE, B_PER_CORE)],
                        tmp16_v)
        bias_vec = bias_v[...]

        @pl.loop(0, B_PER_TILE // L)
        def _acc(j):
            acc = bias_vec
            for t in range(NS):
                acc = acc + tmp16_v[pl.ds(t * B_PER_TILE + j * L, L)]
            out_v[pl.ds(j * L, L)] = acc

    with jax.named_scope("out_dma"):
        pltpu.sync_copy(out_v,
                        out.at[pl.ds(base_b + s * B_PER_TILE, B_PER_TILE)])


_sc_kernel = functools.partial(
    pl.kernel,
    out_type=jax.ShapeDtypeStruct((BATCH,), jnp.float32),
    mesh=plsc.VectorSubcoreMesh(core_axis_name="c", subcore_axis_name="s",
                                num_cores=NC, num_subcores=NS),
    scratch_types=[
        pltpu.VMEM((TAB_ROWS, 128), jnp.float32),
        pltpu.VMEM((TAB_ROWS, 128), jnp.float32),
        pltpu.VMEM((B_PER_CORE,), jnp.int32),
        pltpu.VMEM((B_PER_CORE,), jnp.int32),
        pltpu.VMEM((B_PER_CORE,), jnp.float32),
        pltpu.VMEM((B_PER_CORE,), jnp.float32),
        pltpu.VMEM((B_PER_TILE,), jnp.float32),
        pltpu.VMEM((L,), jnp.float32),
        pltpu.VMEM_SHARED((NS * B_PER_CORE,), jnp.float32),
        pltpu.SemaphoreType.DMA,
        pltpu.SemaphoreType.DMA,
        pltpu.SemaphoreType.DMA,
        pltpu.SemaphoreType.DMA,
        pltpu.SemaphoreType.DMA,
    ],
    compiler_params=pltpu.CompilerParams(needs_layout_passes=False),
)(_sc_body)


@jax.jit
def kernel(x, fc_weight, bias):
    xT = x.astype(jnp.int32).T
    fcp = jnp.pad(fc_weight.astype(jnp.float32), ((0, 384), (0, 0)))
    table = fcp.reshape(TAB_NROW, 128)
    bias16 = jnp.broadcast_to(bias.astype(jnp.float32), (L,))
    out = _sc_kernel(xT, table, bias16)
    return out.reshape(BATCH, 1)


# speedup vs baseline: 1.4215x; 1.0329x over previous
import functools

import jax
import jax.numpy as jnp
from jax import lax
from jax.experimental import pallas as pl
from jax.experimental.pallas import tpu as pltpu
from jax.experimental.pallas import tpu_sc as plsc

NUM_FIELDS = 26
FIELD_DIM = 40000
BATCH = 16384
NC = 2
NS = 16
B_PER_CORE = BATCH // NC
B_PER_TILE = B_PER_CORE // NS
L = 16
TAB_NROW = 8128
TAB_ROWS = 320


def _tab_window(f):
    flat0 = f * FIELD_DIM
    row_start = jnp.minimum((flat0 >> 10) << 3, TAB_NROW - TAB_ROWS)
    row_start = pl.multiple_of(row_start, 8)
    rem = flat0 - (row_start << 7)
    return row_start, rem


def _sc_body(xT, table, bias, out, tab_v, tab2_v, idx1_v, idx2_v, part_v,
             tmp16_v, out_v, bias_v, shared_p, sem_i1, sem_i2, sem_t, sem_t2,
             sem_p):
    c = lax.axis_index("c")
    s = lax.axis_index("s")
    base_b = c * B_PER_CORE
    two = s < NUM_FIELDS - NS
    f1 = s
    f2 = jnp.minimum(s + NS, NUM_FIELDS - 1)

    with jax.named_scope("fire_dmas"):
        d_i1 = pltpu.async_copy(xT.at[f1, pl.ds(base_b, B_PER_CORE)], idx1_v,
                                sem_i1)
        row1, rem1 = _tab_window(f1)
        d_t1 = pltpu.async_copy(table.at[pl.ds(row1, TAB_ROWS)], tab_v, sem_t)
        pltpu.sync_copy(bias, bias_v)
        d_i1.wait()
        d_t1.wait()
        row2, rem2 = _tab_window(f2)

        @pl.when(two)
        def _fire_second():
            pltpu.async_copy(xT.at[f2, pl.ds(base_b, B_PER_CORE)], idx2_v,
                             sem_i2)
            pltpu.async_copy(table.at[pl.ds(row2, TAB_ROWS)], tab2_v, sem_t2)

    with jax.named_scope("gather1"):
        @plsc.parallel_loop(0, B_PER_CORE // L, unroll=2)
        def _gather1(j):
            sl = pl.ds(j * L, L)
            w = idx1_v[sl] + rem1
            part_v[sl] = plsc.load_gather(tab_v, [w >> 7, w & 127])

    @pl.when(two)
    def _second_field():
        with jax.named_scope("drain2"):
            pltpu.make_async_copy(xT.at[f2, pl.ds(base_b, B_PER_CORE)],
                                  idx2_v, sem_i2).wait()
            pltpu.make_async_copy(table.at[pl.ds(row2, TAB_ROWS)], tab2_v,
                                  sem_t2).wait()
        with jax.named_scope("gather2"):
            @plsc.parallel_loop(0, B_PER_CORE // L, unroll=2)
            def _gather2(j):
                sl = pl.ds(j * L, L)
                w = idx2_v[sl] + rem2
                plsc.addupdate(part_v.at[sl],
                               plsc.load_gather(tab2_v, [w >> 7, w & 127]))

    with jax.named_scope("publish"):
        descs = []
        for r in range(NS):
            descs.append(pltpu.async_copy(
                part_v.at[pl.ds(r * B_PER_TILE, B_PER_TILE)],
                shared_p.at[pl.ds(r * B_PER_CORE + s * B_PER_TILE,
                                  B_PER_TILE)],
                sem_p))
        for d in descs:
            d.wait()
    with jax.named_scope("barrier"):
        plsc.subcore_barrier()

    with jax.named_scope("reduce"):
        pltpu.sync_copy(shared_p.at[pl.ds(s * B_PER_CORE, B_PER_CORE)],
                        tmp16_v)
        bias_vec = bias_v[...]

        @pl.loop(0, B_PER_TILE // L)
        def _acc(j):
            acc = bias_vec
            for t in range(NS):
                acc = acc + tmp16_v[pl.ds(t * B_PER_TILE + j * L, L)]
            out_v[pl.ds(j * L, L)] = acc

    with jax.named_scope("out_dma"):
        pltpu.sync_copy(out_v,
                        out.at[pl.ds(base_b + s * B_PER_TILE, B_PER_TILE)])


_sc_kernel = functools.partial(
    pl.kernel,
    out_type=jax.ShapeDtypeStruct((BATCH,), jnp.float32),
    mesh=plsc.VectorSubcoreMesh(core_axis_name="c", subcore_axis_name="s",
                                num_cores=NC, num_subcores=NS),
    scratch_types=[
        pltpu.VMEM((TAB_ROWS, 128), jnp.float32),
        pltpu.VMEM((TAB_ROWS, 128), jnp.float32),
        pltpu.VMEM((B_PER_CORE,), jnp.int32),
        pltpu.VMEM((B_PER_CORE,), jnp.int32),
        pltpu.VMEM((B_PER_CORE,), jnp.float32),
        pltpu.VMEM((B_PER_CORE,), jnp.float32),
        pltpu.VMEM((B_PER_TILE,), jnp.float32),
        pltpu.VMEM((L,), jnp.float32),
        pltpu.VMEM_SHARED((NS * B_PER_CORE,), jnp.float32),
        pltpu.SemaphoreType.DMA,
        pltpu.SemaphoreType.DMA,
        pltpu.SemaphoreType.DMA,
        pltpu.SemaphoreType.DMA,
        pltpu.SemaphoreType.DMA,
    ],
    compiler_params=pltpu.CompilerParams(needs_layout_passes=False),
)(_sc_body)


@jax.jit
def kernel(x, fc_weight, bias):
    xT = x.astype(jnp.int32).T
    fcp = jnp.pad(fc_weight.astype(jnp.float32), ((0, 384), (0, 0)))
    table = fcp.reshape(TAB_NROW, 128)
    bias16 = jnp.broadcast_to(bias.astype(jnp.float32), (L,))
    out = _sc_kernel(xT, table, bias16)
    return out.reshape(BATCH, 1)


# speedup vs baseline: 1.4371x; 1.0109x over previous
import functools

import jax
import jax.numpy as jnp
from jax import lax
from jax.experimental import pallas as pl
from jax.experimental.pallas import tpu as pltpu
from jax.experimental.pallas import tpu_sc as plsc

NUM_FIELDS = 26
FIELD_DIM = 40000
BATCH = 16384
NC = 2
NS = 16
B_PER_CORE = BATCH // NC
B_PER_TILE = B_PER_CORE // NS
L = 16
TAB_NROW = 8128
TAB_ROWS = 320


def _tab_window(f):
    flat0 = f * FIELD_DIM
    row_start = jnp.minimum((flat0 >> 10) << 3, TAB_NROW - TAB_ROWS)
    row_start = pl.multiple_of(row_start, 8)
    rem = flat0 - (row_start << 7)
    return row_start, rem


def _sc_body(xT, table, bias, out, tab_v, tab2_v, idx1_v, idx2_v, part_v,
             tmp16_v, out_v, bias_v, shared_p, sem_i1, sem_i2, sem_t, sem_t2,
             sem_p):
    c = lax.axis_index("c")
    s = lax.axis_index("s")
    base_b = c * B_PER_CORE
    two = s < NUM_FIELDS - NS
    f1 = s
    f2 = jnp.minimum(s + NS, NUM_FIELDS - 1)

    with jax.named_scope("fire_dmas"):
        d_i1 = pltpu.async_copy(xT.at[f1, pl.ds(base_b, B_PER_CORE)], idx1_v,
                                sem_i1)
        row1, rem1 = _tab_window(f1)
        d_t1 = pltpu.async_copy(table.at[pl.ds(row1, TAB_ROWS)], tab_v, sem_t)
        pltpu.sync_copy(bias, bias_v)
        d_i1.wait()
        d_t1.wait()
        row2, rem2 = _tab_window(f2)

        @pl.when(two)
        def _fire_second():
            pltpu.async_copy(xT.at[f2, pl.ds(base_b, B_PER_CORE)], idx2_v,
                             sem_i2)
            pltpu.async_copy(table.at[pl.ds(row2, TAB_ROWS)], tab2_v, sem_t2)

    with jax.named_scope("gather1"):
        @plsc.parallel_loop(0, B_PER_CORE // L, unroll=4)
        def _gather1(j):
            sl = pl.ds(j * L, L)
            w = idx1_v[sl] + rem1
            part_v[sl] = plsc.load_gather(tab_v, [w >> 7, w & 127])

    @pl.when(two)
    def _second_field():
        with jax.named_scope("drain2"):
            pltpu.make_async_copy(xT.at[f2, pl.ds(base_b, B_PER_CORE)],
                                  idx2_v, sem_i2).wait()
            pltpu.make_async_copy(table.at[pl.ds(row2, TAB_ROWS)], tab2_v,
                                  sem_t2).wait()
        with jax.named_scope("gather2"):
            @plsc.parallel_loop(0, B_PER_CORE // L, unroll=4)
            def _gather2(j):
                sl = pl.ds(j * L, L)
                w = idx2_v[sl] + rem2
                plsc.addupdate(part_v.at[sl],
                               plsc.load_gather(tab2_v, [w >> 7, w & 127]))

    with jax.named_scope("publish"):
        descs = []
        for r in range(NS):
            descs.append(pltpu.async_copy(
                part_v.at[pl.ds(r * B_PER_TILE, B_PER_TILE)],
                shared_p.at[pl.ds(r * B_PER_CORE + s * B_PER_TILE,
                                  B_PER_TILE)],
                sem_p))
        for d in descs:
            d.wait()
    with jax.named_scope("barrier"):
        plsc.subcore_barrier()

    with jax.named_scope("reduce"):
        pltpu.sync_copy(shared_p.at[pl.ds(s * B_PER_CORE, B_PER_CORE)],
                        tmp16_v)
        bias_vec = bias_v[...]

        @plsc.parallel_loop(0, B_PER_TILE // L, unroll=2)
        def _acc(j):
            acc = bias_vec
            for t in range(NS):
                acc = acc + tmp16_v[pl.ds(t * B_PER_TILE + j * L, L)]
            out_v[pl.ds(j * L, L)] = acc

    with jax.named_scope("out_dma"):
        pltpu.sync_copy(out_v,
                        out.at[pl.ds(base_b + s * B_PER_TILE, B_PER_TILE)])


_sc_kernel = functools.partial(
    pl.kernel,
    out_type=jax.ShapeDtypeStruct((BATCH,), jnp.float32),
    mesh=plsc.VectorSubcoreMesh(core_axis_name="c", subcore_axis_name="s",
                                num_cores=NC, num_subcores=NS),
    scratch_types=[
        pltpu.VMEM((TAB_ROWS, 128), jnp.float32),
        pltpu.VMEM((TAB_ROWS, 128), jnp.float32),
        pltpu.VMEM((B_PER_CORE,), jnp.int32),
        pltpu.VMEM((B_PER_CORE,), jnp.int32),
        pltpu.VMEM((B_PER_CORE,), jnp.float32),
        pltpu.VMEM((B_PER_CORE,), jnp.float32),
        pltpu.VMEM((B_PER_TILE,), jnp.float32),
        pltpu.VMEM((L,), jnp.float32),
        pltpu.VMEM_SHARED((NS * B_PER_CORE,), jnp.float32),
        pltpu.SemaphoreType.DMA,
        pltpu.SemaphoreType.DMA,
        pltpu.SemaphoreType.DMA,
        pltpu.SemaphoreType.DMA,
        pltpu.SemaphoreType.DMA,
    ],
    compiler_params=pltpu.CompilerParams(needs_layout_passes=False),
)(_sc_body)


@jax.jit
def kernel(x, fc_weight, bias):
    xT = x.astype(jnp.int32).T
    fcp = jnp.pad(fc_weight.astype(jnp.float32), ((0, 384), (0, 0)))
    table = fcp.reshape(TAB_NROW, 128)
    bias16 = jnp.broadcast_to(bias.astype(jnp.float32), (L,))
    out = _sc_kernel(xT, table, bias16)
    return out.reshape(BATCH, 1)


# speedup vs baseline: 1.4425x; 1.0038x over previous
import functools

import jax
import jax.numpy as jnp
from jax import lax
from jax.experimental import pallas as pl
from jax.experimental.pallas import tpu as pltpu
from jax.experimental.pallas import tpu_sc as plsc

NUM_FIELDS = 26
FIELD_DIM = 40000
BATCH = 16384
NC = 2
NS = 16
B_PER_CORE = BATCH // NC
B_PER_TILE = B_PER_CORE // NS
L = 16
TAB_NROW = 8128
TAB_ROWS = 320


def _tab_window(f):
    flat0 = f * FIELD_DIM
    row_start = jnp.minimum((flat0 >> 10) << 3, TAB_NROW - TAB_ROWS)
    row_start = pl.multiple_of(row_start, 8)
    rem = flat0 - (row_start << 7)
    return row_start, rem


def _sc_body(xT, table, bias, out, tab_v, tab2_v, idx1_v, idx2_v, part_v,
             tmp16_v, out_v, bias_v, shared_p, sem_i1, sem_i2, sem_t, sem_t2,
             sem_p):
    c = lax.axis_index("c")
    s = lax.axis_index("s")
    base_b = c * B_PER_CORE
    two = s < NUM_FIELDS - NS
    f1 = s
    f2 = jnp.minimum(s + NS, NUM_FIELDS - 1)

    with jax.named_scope("fire_dmas"):
        d_i1 = pltpu.async_copy(xT.at[f1, pl.ds(base_b, B_PER_CORE)], idx1_v,
                                sem_i1)
        row1, rem1 = _tab_window(f1)
        d_t1 = pltpu.async_copy(table.at[pl.ds(row1, TAB_ROWS)], tab_v, sem_t)
        pltpu.sync_copy(bias, bias_v)
        d_i1.wait()
        d_t1.wait()
        row2, rem2 = _tab_window(f2)

        @pl.when(two)
        def _fire_second():
            pltpu.async_copy(xT.at[f2, pl.ds(base_b, B_PER_CORE)], idx2_v,
                             sem_i2)
            pltpu.async_copy(table.at[pl.ds(row2, TAB_ROWS)], tab2_v, sem_t2)

    with jax.named_scope("gather1"):
        @plsc.parallel_loop(0, B_PER_CORE // L, unroll=8)
        def _gather1(j):
            sl = pl.ds(j * L, L)
            w = idx1_v[sl] + rem1
            part_v[sl] = plsc.load_gather(tab_v, [w >> 7, w & 127])

    @pl.when(two)
    def _second_field():
        with jax.named_scope("drain2"):
            pltpu.make_async_copy(xT.at[f2, pl.ds(base_b, B_PER_CORE)],
                                  idx2_v, sem_i2).wait()
            pltpu.make_async_copy(table.at[pl.ds(row2, TAB_ROWS)], tab2_v,
                                  sem_t2).wait()
        with jax.named_scope("gather2"):
            @plsc.parallel_loop(0, B_PER_CORE // L, unroll=8)
            def _gather2(j):
                sl = pl.ds(j * L, L)
                w = idx2_v[sl] + rem2
                plsc.addupdate(part_v.at[sl],
                               plsc.load_gather(tab2_v, [w >> 7, w & 127]))

    with jax.named_scope("publish"):
        descs = []
        for r in range(NS):
            descs.append(pltpu.async_copy(
                part_v.at[pl.ds(r * B_PER_TILE, B_PER_TILE)],
                shared_p.at[pl.ds(r * B_PER_CORE + s * B_PER_TILE,
                                  B_PER_TILE)],
                sem_p))
        for d in descs:
            d.wait()
    with jax.named_scope("barrier"):
        plsc.subcore_barrier()

    with jax.named_scope("reduce"):
        pltpu.sync_copy(shared_p.at[pl.ds(s * B_PER_CORE, B_PER_CORE)],
                        tmp16_v)
        bias_vec = bias_v[...]

        @plsc.parallel_loop(0, B_PER_TILE // L, unroll=2)
        def _acc(j):
            acc = bias_vec
            for t in range(NS):
                acc = acc + tmp16_v[pl.ds(t * B_PER_TILE + j * L, L)]
            out_v[pl.ds(j * L, L)] = acc

    with jax.named_scope("out_dma"):
        pltpu.sync_copy(out_v,
                        out.at[pl.ds(base_b + s * B_PER_TILE, B_PER_TILE)])


_sc_kernel = functools.partial(
    pl.kernel,
    out_type=jax.ShapeDtypeStruct((BATCH,), jnp.float32),
    mesh=plsc.VectorSubcoreMesh(core_axis_name="c", subcore_axis_name="s",
                                num_cores=NC, num_subcores=NS),
    scratch_types=[
        pltpu.VMEM((TAB_ROWS, 128), jnp.float32),
        pltpu.VMEM((TAB_ROWS, 128), jnp.float32),
        pltpu.VMEM((B_PER_CORE,), jnp.int32),
        pltpu.VMEM((B_PER_CORE,), jnp.int32),
        pltpu.VMEM((B_PER_CORE,), jnp.float32),
        pltpu.VMEM((B_PER_CORE,), jnp.float32),
        pltpu.VMEM((B_PER_TILE,), jnp.float32),
        pltpu.VMEM((L,), jnp.float32),
        pltpu.VMEM_SHARED((NS * B_PER_CORE,), jnp.float32),
        pltpu.SemaphoreType.DMA,
        pltpu.SemaphoreType.DMA,
        pltpu.SemaphoreType.DMA,
        pltpu.SemaphoreType.DMA,
        pltpu.SemaphoreType.DMA,
    ],
    compiler_params=pltpu.CompilerParams(needs_layout_passes=False),
)(_sc_body)


@jax.jit
def kernel(x, fc_weight, bias):
    xT = x.astype(jnp.int32).T
    fcp = jnp.pad(fc_weight.astype(jnp.float32), ((0, 384), (0, 0)))
    table = fcp.reshape(TAB_NROW, 128)
    bias16 = jnp.broadcast_to(bias.astype(jnp.float32), (L,))
    out = _sc_kernel(xT, table, bias16)
    return out.reshape(BATCH, 1)
